# baseline (speedup 1.0000x reference)
import functools

import jax
import jax.numpy as jnp
from jax import lax
from jax.experimental import pallas as pl
from jax.experimental.pallas import tpu as pltpu
from jax.experimental.pallas import tpu_sc as plsc

_N = 10000
_E = 320000
_H = 128
_ED = 16

_NC = 2
_NS = 16
_NW = _NC * _NS
_EPW = _E // _NW
_C = 80
_NCH = _EPW // _C
_NBUF = 4

_f32 = jnp.float32
_bf16 = jnp.bfloat16
_HP = _H // 2


def _sc_gather(table, dst3, srcn3):
    mesh = plsc.VectorSubcoreMesh(core_axis_name="c", subcore_axis_name="s")

    @functools.partial(
        pl.kernel,
        mesh=mesh,
        compiler_params=pltpu.CompilerParams(use_tc_tiling_on_sc=False),
        out_type=(
            jax.ShapeDtypeStruct((_E, _HP), jnp.int32),
            jax.ShapeDtypeStruct((_E, _HP), jnp.int32),
        ),
        scratch_types=[
            pltpu.VMEM((_NCH, 1, _C), jnp.int32),
            pltpu.VMEM((_NCH, 1, _C), jnp.int32),
            pltpu.VMEM((_NBUF, _C, _HP), jnp.int32),
            pltpu.VMEM((_NBUF, _C, _HP), jnp.int32),
        ] + [pltpu.SemaphoreType.DMA] * (4 * _NBUF),
    )
    def k(t_hbm, dst_hbm, srcn_hbm, pr_hbm, qr_hbm, di, si, pbuf, qbuf, *sems):
        gp = sems[0:_NBUF]
        gq = sems[_NBUF:2 * _NBUF]
        wp = sems[2 * _NBUF:3 * _NBUF]
        wq = sems[3 * _NBUF:4 * _NBUF]
        wid = lax.axis_index("s") * _NC + lax.axis_index("c")
        pltpu.sync_copy(dst_hbm.at[wid], di)
        pltpu.sync_copy(srcn_hbm.at[wid], si)

        def issue_gather(j, b):
            pltpu.async_copy(t_hbm.at[di.at[j, 0]], pbuf.at[b], gp[b])
            pltpu.async_copy(t_hbm.at[si.at[j, 0]], qbuf.at[b], gq[b])

        def rows(j):
            return pl.ds(wid * _EPW + j * _C, _C)

        issue_gather(0, 0)
        issue_gather(1, 1)

        def step(i, carry):
            for b in range(_NBUF):
                j = i * _NBUF + b
                ba = (b + 2) % _NBUF

                @pl.when((j >= 2) & (j < _NCH + 2))
                def _():
                    pltpu.make_async_copy(pbuf.at[ba], pr_hbm.at[rows(j - 2)],
                                          wp[ba]).wait()
                    pltpu.make_async_copy(qbuf.at[ba], qr_hbm.at[rows(j - 2)],
                                          wq[ba]).wait()

                @pl.when(j + 2 < _NCH)
                def _():
                    issue_gather(j + 2, ba)

                @pl.when(j < _NCH)
                def _():
                    pltpu.make_async_copy(t_hbm.at[di.at[j, 0]], pbuf.at[b],
                                          gp[b]).wait()
                    pltpu.make_async_copy(t_hbm.at[si.at[j, 0]], qbuf.at[b],
                                          gq[b]).wait()
                    pltpu.async_copy(pbuf.at[b], pr_hbm.at[rows(j)], wp[b])
                    pltpu.async_copy(qbuf.at[b], qr_hbm.at[rows(j)], wq[b])
            return carry

        lax.fori_loop(0, (_NCH + 2 + _NBUF - 1) // _NBUF, step, 0)

    return k(table, dst3, srcn3)


def _sc_scatter(mg, dst3, zeros):
    mesh = plsc.VectorSubcoreMesh(core_axis_name="c", subcore_axis_name="s")

    nbuf = 4

    @functools.partial(
        pl.kernel,
        mesh=mesh,
        out_type=jax.ShapeDtypeStruct((2 * _N, _H), _f32),
        scratch_types=[
            pltpu.VMEM_SHARED((_N, _H), _f32),
            pltpu.VMEM((nbuf, 1, _C), jnp.int32),
            pltpu.VMEM((nbuf, _C, _H), _f32),
        ] + [pltpu.SemaphoreType.DMA] * (3 * nbuf),
    )
    def k(mg_hbm, dst_hbm, z_hbm, out_hbm, acc_sh, ibuf, mbuf, *sems):
        rd = sems[0:nbuf]
        ri = sems[nbuf:2 * nbuf]
        sc = sems[2 * nbuf:3 * nbuf]
        c = lax.axis_index("c")
        s = lax.axis_index("s")
        wid = s * _NC + c

        @pl.when(s < 10)
        def _():
            pltpu.sync_copy(z_hbm.at[pl.ds(s * 1000, 1000)],
                            acc_sh.at[pl.ds(s * 1000, 1000)])

        plsc.subcore_barrier()

        def rows(j):
            return pl.ds(wid * _EPW + j * _C, _C)

        def issue_read(j, b):
            pltpu.async_copy(dst_hbm.at[wid, j], ibuf.at[b], ri[b])
            pltpu.async_copy(mg_hbm.at[rows(j)], mbuf.at[b], rd[b])

        issue_read(0, 0)
        issue_read(1, 1)

        def step(i, carry):
            for b in range(nbuf):
                j = i * nbuf + b
                ba = (b + 2) % nbuf

                @pl.when((j >= 2) & (j < _NCH + 2))
                def _():
                    pltpu.make_async_copy(mbuf.at[ba],
                                          acc_sh.at[ibuf.at[ba, 0]],
                                          sc[ba]).wait()

                @pl.when(j + 2 < _NCH)
                def _():
                    issue_read(j + 2, ba)

                @pl.when(j < _NCH)
                def _():
                    pltpu.make_async_copy(mg_hbm.at[rows(j)], mbuf.at[b],
                                          rd[b]).wait()
                    pltpu.make_async_copy(dst_hbm.at[wid, j], ibuf.at[b],
                                          ri[b]).wait()
                    pltpu.async_copy(mbuf.at[b], acc_sh.at[ibuf.at[b, 0]],
                                     sc[b], add=True)
            return carry

        lax.fori_loop(0, (_NCH + 2 + nbuf - 1) // nbuf, step, 0)
        plsc.subcore_barrier()

        @pl.when(s < 10)
        def _():
            pltpu.sync_copy(acc_sh.at[pl.ds(s * 1000, 1000)],
                            out_hbm.at[pl.ds(c * _N + s * 1000, 1000)])

    return k(mg, dst3, zeros)


def _tc_project(h, w_stack):
    bn = 1000

    def body(h_ref, w_ref, o_ref):
        o_ref[...] = jnp.dot(h_ref[...], w_ref[0],
                             preferred_element_type=_f32).astype(_bf16)

    return pl.pallas_call(
        body,
        grid=(2 * _N // bn,),
        in_specs=[
            pl.BlockSpec((bn, _H), lambda g: (g % (_N // bn), 0)),
            pl.BlockSpec((1, _H, _H), lambda g: (g // (_N // bn), 0, 0)),
        ],
        out_specs=pl.BlockSpec((bn, _H), lambda g: (g, 0)),
        out_shape=jax.ShapeDtypeStruct((2 * _N, _H), _bf16),
    )(h, w_stack)


def _tc_edge_mlp(attr, pr, qr, wa, b1, w2, b2, iw, ib):
    be = 2000

    def unpack(xi):
        lo = lax.bitcast_convert_type(lax.shift_left(xi, 16), _f32)
        hi = lax.bitcast_convert_type(lax.bitwise_and(xi, jnp.int32(-65536)),
                                      _f32)
        return jnp.concatenate([lo, hi], axis=1)

    def body(a_ref, p_ref, q_ref, wa_ref, b1_ref, w2_ref, b2_ref, iw_ref,
             ib_ref, o_ref):
        x = (jnp.dot(a_ref[...], wa_ref[...], preferred_element_type=_f32)
             + unpack(p_ref[...]) + unpack(q_ref[...]) + b1_ref[...])
        m = jnp.maximum(x, 0.0)
        mij = jnp.maximum(
            jnp.dot(m, w2_ref[...], preferred_element_type=_f32)
            + b2_ref[...], 0.0)
        t = jnp.sum(mij * iw_ref[...], axis=1, keepdims=True) + ib_ref[0, 0]
        eij = 1.0 / (1.0 + jnp.exp(-t))
        o_ref[...] = mij * eij

    full = lambda g: (0, 0)
    return pl.pallas_call(
        body,
        grid=(_E // be,),
        in_specs=[
            pl.BlockSpec((be, _ED), lambda g: (g, 0)),
            pl.BlockSpec((be, _HP), lambda g: (g, 0)),
            pl.BlockSpec((be, _HP), lambda g: (g, 0)),
            pl.BlockSpec((_ED, _H), full),
            pl.BlockSpec((1, _H), full),
            pl.BlockSpec((_H, _H), full),
            pl.BlockSpec((1, _H), full),
            pl.BlockSpec((1, _H), full),
            pl.BlockSpec((1, 1), full),
        ],
        out_specs=pl.BlockSpec((be, _H), lambda g: (g, 0)),
        out_shape=jax.ShapeDtypeStruct((_E, _H), _f32),
    )(attr, pr, qr, wa, b1, w2, b2, iw, ib)


def _tc_node_mlp(partials, h, wmi, wh, b1, w2, b2):
    bn = 1000

    def body(p0_ref, p1_ref, h_ref, wmi_ref, wh_ref, b1_ref, w2_ref, b2_ref,
             o_ref):
        mi = p0_ref[...] + p1_ref[...]
        z = jnp.maximum(
            jnp.dot(mi, wmi_ref[...], preferred_element_type=_f32)
            + jnp.dot(h_ref[...], wh_ref[...], preferred_element_type=_f32)
            + b1_ref[...], 0.0)
        o_ref[...] = jnp.dot(z, w2_ref[...],
                             preferred_element_type=_f32) + b2_ref[...]

    full = lambda g: (0, 0)
    nb = _N // bn
    return pl.pallas_call(
        body,
        grid=(nb,),
        in_specs=[
            pl.BlockSpec((bn, _H), lambda g: (g, 0)),
            pl.BlockSpec((bn, _H), lambda g: (g + nb, 0)),
            pl.BlockSpec((bn, _H), lambda g: (g, 0)),
            pl.BlockSpec((_H, _H), full),
            pl.BlockSpec((_H, _H), full),
            pl.BlockSpec((1, _H), full),
            pl.BlockSpec((_H, _H), full),
            pl.BlockSpec((1, _H), full),
        ],
        out_specs=pl.BlockSpec((bn, _H), lambda g: (g, 0)),
        out_shape=jax.ShapeDtypeStruct((_N, _H), _f32),
    )(partials, partials, h, wmi, wh, b1, w2, b2)


def kernel(h, edge_index, edge_attr, e_w1, e_b1, e_w2, e_b2, i_w, i_b,
           n_w1, n_b1, n_w2, n_b2):
    dst = edge_index[0].astype(jnp.int32)
    srcn = (edge_index[1] + _N).astype(jnp.int32)
    dst3 = dst.reshape(_NW, _NCH, 1, _C)
    srcn3 = srcn.reshape(_NW, _NCH, 1, _C)

    w_stack = jnp.stack([e_w1[_ED:_ED + _H], e_w1[_ED + _H:]])
    table = _tc_project(h, w_stack)
    tpack = lax.bitcast_convert_type(table.reshape(2 * _N, _HP, 2),
                                     jnp.int32)

    pr, qr = _sc_gather(tpack, dst3, srcn3)

    pi = jnp.concatenate([jnp.arange(0, _H, 2), jnp.arange(1, _H, 2)])
    mg = _tc_edge_mlp(edge_attr, pr, qr,
                      e_w1[:_ED, pi], e_b1[pi].reshape(1, _H),
                      e_w2[pi, :], e_b2.reshape(1, _H),
                      i_w.reshape(1, _H), i_b.reshape(1, 1))

    partials = _sc_scatter(mg, dst3, jnp.zeros((_N, _H), _f32))

    return _tc_node_mlp(partials, h,
                        n_w1[:_H], n_w1[_H:], n_b1.reshape(1, _H),
                        n_w2, n_b2.reshape(1, _H))


# speedup vs baseline: 1.4962x; 1.4962x over previous
import functools

import jax
import jax.numpy as jnp
from jax import lax
from jax.experimental import pallas as pl
from jax.experimental.pallas import tpu as pltpu
from jax.experimental.pallas import tpu_sc as plsc

_N = 10000
_E = 320000
_H = 128
_ED = 16

_NC = 2
_NS = 16
_NW = _NC * _NS
_NBUF = 4

_NE0 = 161280
_NE1 = _E - _NE0
_GC0, _GC1 = 112, 80
_SC0, _SC1 = 80, 80
_BE0, _BE1 = 2016, 2480

_f32 = jnp.float32


def _sc_gather(table, dst3, srcn3, ne, c):
    epw = ne // _NW
    nch = epw // c
    mesh = plsc.VectorSubcoreMesh(core_axis_name="c", subcore_axis_name="s")

    @functools.partial(
        pl.kernel,
        mesh=mesh,
        out_type=(
            jax.ShapeDtypeStruct((ne, _H), _f32),
            jax.ShapeDtypeStruct((ne, _H), _f32),
        ),
        scratch_types=[
            pltpu.VMEM((nch, 1, c), jnp.int32),
            pltpu.VMEM((nch, 1, c), jnp.int32),
            pltpu.VMEM((_NBUF, c, _H), _f32),
            pltpu.VMEM((_NBUF, c, _H), _f32),
        ] + [pltpu.SemaphoreType.DMA] * (4 * _NBUF),
    )
    def k(t_hbm, dst_hbm, srcn_hbm, pr_hbm, qr_hbm, di, si, pbuf, qbuf, *sems):
        gp = sems[0:_NBUF]
        gq = sems[_NBUF:2 * _NBUF]
        wp = sems[2 * _NBUF:3 * _NBUF]
        wq = sems[3 * _NBUF:4 * _NBUF]
        wid = lax.axis_index("s") * _NC + lax.axis_index("c")
        pltpu.sync_copy(dst_hbm.at[wid], di)
        pltpu.sync_copy(srcn_hbm.at[wid], si)

        def issue_gather(j, b):
            pltpu.async_copy(t_hbm.at[di.at[j, 0]], pbuf.at[b], gp[b])
            pltpu.async_copy(t_hbm.at[si.at[j, 0]], qbuf.at[b], gq[b])

        def rows(j):
            return pl.ds(wid * epw + j * c, c)

        issue_gather(0, 0)
        issue_gather(1, 1)

        def step(i, carry):
            for b in range(_NBUF):
                j = i * _NBUF + b
                ba = (b + 2) % _NBUF

                @pl.when((j >= 2) & (j < nch + 2))
                def _():
                    pltpu.make_async_copy(pbuf.at[ba], pr_hbm.at[rows(j - 2)],
                                          wp[ba]).wait()
                    pltpu.make_async_copy(qbuf.at[ba], qr_hbm.at[rows(j - 2)],
                                          wq[ba]).wait()

                @pl.when(j + 2 < nch)
                def _():
                    issue_gather(j + 2, ba)

                @pl.when(j < nch)
                def _():
                    pltpu.make_async_copy(t_hbm.at[di.at[j, 0]], pbuf.at[b],
                                          gp[b]).wait()
                    pltpu.make_async_copy(t_hbm.at[si.at[j, 0]], qbuf.at[b],
                                          gq[b]).wait()
                    pltpu.async_copy(pbuf.at[b], pr_hbm.at[rows(j)], wp[b])
                    pltpu.async_copy(qbuf.at[b], qr_hbm.at[rows(j)], wq[b])
            return carry

        lax.fori_loop(0, (nch + 2 + _NBUF - 1) // _NBUF, step, 0)

    return k(table, dst3, srcn3)


def _sc_scatter(mg, dst3, zeros, ne, c):
    epw = ne // _NW
    nch = epw // c
    mesh = plsc.VectorSubcoreMesh(core_axis_name="c", subcore_axis_name="s")

    nbuf = 4

    @functools.partial(
        pl.kernel,
        mesh=mesh,
        out_type=jax.ShapeDtypeStruct((2 * _N, _H), _f32),
        scratch_types=[
            pltpu.VMEM_SHARED((_N, _H), _f32),
            pltpu.VMEM((nbuf, 1, c), jnp.int32),
            pltpu.VMEM((nbuf, c, _H), _f32),
        ] + [pltpu.SemaphoreType.DMA] * (3 * nbuf),
    )
    def k(mg_hbm, dst_hbm, z_hbm, out_hbm, acc_sh, ibuf, mbuf, *sems):
        rd = sems[0:nbuf]
        ri = sems[nbuf:2 * nbuf]
        sc = sems[2 * nbuf:3 * nbuf]
        cc = lax.axis_index("c")
        s = lax.axis_index("s")
        wid = s * _NC + cc

        @pl.when(s < 10)
        def _():
            pltpu.sync_copy(z_hbm.at[pl.ds(s * 1000, 1000)],
                            acc_sh.at[pl.ds(s * 1000, 1000)])

        plsc.subcore_barrier()

        def rows(j):
            return pl.ds(wid * epw + j * c, c)

        def issue_read(j, b):
            pltpu.async_copy(dst_hbm.at[wid, j], ibuf.at[b], ri[b])
            pltpu.async_copy(mg_hbm.at[rows(j)], mbuf.at[b], rd[b])

        issue_read(0, 0)
        issue_read(1, 1)

        def step(i, carry):
            for b in range(nbuf):
                j = i * nbuf + b
                ba = (b + 2) % nbuf

                @pl.when((j >= 2) & (j < nch + 2))
                def _():
                    pltpu.make_async_copy(mbuf.at[ba],
                                          acc_sh.at[ibuf.at[ba, 0]],
                                          sc[ba]).wait()

                @pl.when(j + 2 < nch)
                def _():
                    issue_read(j + 2, ba)

                @pl.when(j < nch)
                def _():
                    pltpu.make_async_copy(mg_hbm.at[rows(j)], mbuf.at[b],
                                          rd[b]).wait()
                    pltpu.make_async_copy(dst_hbm.at[wid, j], ibuf.at[b],
                                          ri[b]).wait()
                    pltpu.async_copy(mbuf.at[b], acc_sh.at[ibuf.at[b, 0]],
                                     sc[b], add=True)
            return carry

        lax.fori_loop(0, (nch + 2 + nbuf - 1) // nbuf, step, 0)
        plsc.subcore_barrier()

        @pl.when(s < 10)
        def _():
            pltpu.sync_copy(acc_sh.at[pl.ds(s * 1000, 1000)],
                            out_hbm.at[pl.ds(cc * _N + s * 1000, 1000)])

    return k(mg, dst3, zeros)


def _tc_project(h, w_stack):
    bn = 1000

    def body(h_ref, w_ref, o_ref):
        o_ref[...] = jnp.dot(h_ref[...], w_ref[0],
                             preferred_element_type=_f32)

    return pl.pallas_call(
        body,
        grid=(2 * _N // bn,),
        in_specs=[
            pl.BlockSpec((bn, _H), lambda g: (g % (_N // bn), 0)),
            pl.BlockSpec((1, _H, _H), lambda g: (g // (_N // bn), 0, 0)),
        ],
        out_specs=pl.BlockSpec((bn, _H), lambda g: (g, 0)),
        out_shape=jax.ShapeDtypeStruct((2 * _N, _H), _f32),
    )(h, w_stack)


def _tc_edge_mlp(attr, pr, qr, wa, b1, w2, b2, iw, ib, ne, be):
    def body(a_ref, p_ref, q_ref, wa_ref, b1_ref, w2_ref, b2_ref, iw_ref,
             ib_ref, o_ref):
        x = (jnp.dot(a_ref[...], wa_ref[...], preferred_element_type=_f32)
             + p_ref[...] + q_ref[...] + b1_ref[...])
        m = jnp.maximum(x, 0.0)
        mij = jnp.maximum(
            jnp.dot(m, w2_ref[...], preferred_element_type=_f32)
            + b2_ref[...], 0.0)
        t = jnp.sum(mij * iw_ref[...], axis=1, keepdims=True) + ib_ref[0, 0]
        eij = 1.0 / (1.0 + jnp.exp(-t))
        o_ref[...] = mij * eij

    full = lambda g: (0, 0)
    return pl.pallas_call(
        body,
        grid=(ne // be,),
        in_specs=[
            pl.BlockSpec((be, _ED), lambda g: (g, 0)),
            pl.BlockSpec((be, _H), lambda g: (g, 0)),
            pl.BlockSpec((be, _H), lambda g: (g, 0)),
            pl.BlockSpec((_ED, _H), full),
            pl.BlockSpec((1, _H), full),
            pl.BlockSpec((_H, _H), full),
            pl.BlockSpec((1, _H), full),
            pl.BlockSpec((1, _H), full),
            pl.BlockSpec((1, 1), full),
        ],
        out_specs=pl.BlockSpec((be, _H), lambda g: (g, 0)),
        out_shape=jax.ShapeDtypeStruct((ne, _H), _f32),
    )(attr, pr, qr, wa, b1, w2, b2, iw, ib)


def _tc_node_mlp(part_a, part_b, h, wmi, wh, b1, w2, b2):
    bn = 1000

    def body(pa0_ref, pa1_ref, pb0_ref, pb1_ref, h_ref, wmi_ref, wh_ref,
             b1_ref, w2_ref, b2_ref, o_ref):
        mi = (pa0_ref[...] + pa1_ref[...]) + (pb0_ref[...] + pb1_ref[...])
        z = jnp.maximum(
            jnp.dot(mi, wmi_ref[...], preferred_element_type=_f32)
            + jnp.dot(h_ref[...], wh_ref[...], preferred_element_type=_f32)
            + b1_ref[...], 0.0)
        o_ref[...] = jnp.dot(z, w2_ref[...],
                             preferred_element_type=_f32) + b2_ref[...]

    full = lambda g: (0, 0)
    nb = _N // bn
    return pl.pallas_call(
        body,
        grid=(nb,),
        in_specs=[
            pl.BlockSpec((bn, _H), lambda g: (g, 0)),
            pl.BlockSpec((bn, _H), lambda g: (g + nb, 0)),
            pl.BlockSpec((bn, _H), lambda g: (g, 0)),
            pl.BlockSpec((bn, _H), lambda g: (g + nb, 0)),
            pl.BlockSpec((bn, _H), lambda g: (g, 0)),
            pl.BlockSpec((_H, _H), full),
            pl.BlockSpec((_H, _H), full),
            pl.BlockSpec((1, _H), full),
            pl.BlockSpec((_H, _H), full),
            pl.BlockSpec((1, _H), full),
        ],
        out_specs=pl.BlockSpec((bn, _H), lambda g: (g, 0)),
        out_shape=jax.ShapeDtypeStruct((_N, _H), _f32),
    )(part_a, part_a, part_b, part_b, h, wmi, wh, b1, w2, b2)


def kernel(h, edge_index, edge_attr, e_w1, e_b1, e_w2, e_b2, i_w, i_b,
           n_w1, n_b1, n_w2, n_b2):
    dst = edge_index[0].astype(jnp.int32)
    srcn = (edge_index[1] + _N).astype(jnp.int32)

    w_stack = jnp.stack([e_w1[_ED:_ED + _H], e_w1[_ED + _H:]])
    table = _tc_project(h, w_stack)

    zeros = jnp.zeros((_N, _H), _f32)
    wa = e_w1[:_ED]
    b1 = e_b1.reshape(1, _H)
    b2 = e_b2.reshape(1, _H)
    iw = i_w.reshape(1, _H)
    ib = i_b.reshape(1, 1)

    halves = []
    for lo, ne, gc, sc in ((0, _NE0, _GC0, _SC0), (_NE0, _NE1, _GC1, _SC1)):
        epw = ne // _NW
        dh = lax.dynamic_slice_in_dim(dst, lo, ne)
        sh = lax.dynamic_slice_in_dim(srcn, lo, ne)
        halves.append({
            "lo": lo, "ne": ne, "gc": gc, "sc": sc,
            "gd3": dh.reshape(_NW, epw // gc, 1, gc),
            "gs3": sh.reshape(_NW, epw // gc, 1, gc),
            "sd3": dh.reshape(_NW, epw // sc, 1, sc),
        })

    for hv in halves:
        hv["pr"], hv["qr"] = _sc_gather(table, hv["gd3"], hv["gs3"],
                                        hv["ne"], hv["gc"])

    for hv, be in zip(halves, (_BE0, _BE1)):
        attr = lax.dynamic_slice_in_dim(edge_attr, hv["lo"], hv["ne"])
        hv["mg"] = _tc_edge_mlp(attr, hv["pr"], hv["qr"], wa, b1, e_w2, b2,
                                iw, ib, hv["ne"], be)

    for hv in halves:
        hv["part"] = _sc_scatter(hv["mg"], hv["sd3"], zeros, hv["ne"],
                                 hv["sc"])

    return _tc_node_mlp(halves[0]["part"], halves[1]["part"], h,
                        n_w1[:_H], n_w1[_H:], n_b1.reshape(1, _H),
                        n_w2, n_b2.reshape(1, _H))


# speedup vs baseline: 1.7274x; 1.1545x over previous
import functools

import jax
import jax.numpy as jnp
from jax import lax
from jax.experimental import pallas as pl
from jax.experimental.pallas import tpu as pltpu
from jax.experimental.pallas import tpu_sc as plsc

_N = 10000
_E = 320000
_H = 128
_ED = 16

_NC = 2
_NS = 16
_NW = _NC * _NS
_NBUF = 4

_NE0 = 161280
_NE1 = _E - _NE0
_GC0, _GC1 = 112, 80
_SC0, _SC1 = 80, 80
_BE0, _BE1 = 2016, 2480

_f32 = jnp.float32


def _sc_gather(table, dst3, srcn3, ne, c):
    epw = ne // _NW
    nch = epw // c
    mesh = plsc.VectorSubcoreMesh(core_axis_name="c", subcore_axis_name="s")

    @functools.partial(
        pl.kernel,
        mesh=mesh,
        out_type=jax.ShapeDtypeStruct((ne, _H), _f32),
        scratch_types=[
            pltpu.VMEM((nch, 1, c), jnp.int32),
            pltpu.VMEM((nch, 1, c), jnp.int32),
            pltpu.VMEM((_NBUF, c, _H), _f32),
            pltpu.VMEM((_NBUF, c, _H), _f32),
        ] + [pltpu.SemaphoreType.DMA] * (3 * _NBUF),
    )
    def k(t_hbm, dst_hbm, srcn_hbm, g_hbm, di, si, pbuf, qbuf, *sems):
        gp = sems[0:_NBUF]
        gq = sems[_NBUF:2 * _NBUF]
        wp = sems[2 * _NBUF:3 * _NBUF]
        wid = lax.axis_index("s") * _NC + lax.axis_index("c")
        pltpu.sync_copy(dst_hbm.at[wid], di)
        pltpu.sync_copy(srcn_hbm.at[wid], si)

        def issue_gather(j, b):
            pltpu.async_copy(t_hbm.at[di.at[j, 0]], pbuf.at[b], gp[b])
            pltpu.async_copy(t_hbm.at[si.at[j, 0]], qbuf.at[b], gq[b])

        def rows(j):
            return pl.ds(wid * epw + j * c, c)

        issue_gather(0, 0)
        issue_gather(1, 1)

        def step(i, carry):
            for b in range(_NBUF):
                j = i * _NBUF + b
                ba = (b + 2) % _NBUF

                @pl.when((j >= 2) & (j < nch + 2))
                def _():
                    pltpu.make_async_copy(pbuf.at[ba], g_hbm.at[rows(j - 2)],
                                          wp[ba]).wait()

                @pl.when(j + 2 < nch)
                def _():
                    issue_gather(j + 2, ba)

                @pl.when(j < nch)
                def _():
                    pltpu.make_async_copy(t_hbm.at[di.at[j, 0]], pbuf.at[b],
                                          gp[b]).wait()
                    pltpu.make_async_copy(t_hbm.at[si.at[j, 0]], qbuf.at[b],
                                          gq[b]).wait()

                    def row(r, rc):
                        for kk in range(_H // 16):
                            sl = pl.ds(kk * 16, 16)
                            pbuf[b, r, sl] = pbuf[b, r, sl] + qbuf[b, r, sl]
                        return rc

                    lax.fori_loop(0, c, row, 0)
                    pltpu.async_copy(pbuf.at[b], g_hbm.at[rows(j)], wp[b])
            return carry

        lax.fori_loop(0, (nch + 2 + _NBUF - 1) // _NBUF, step, 0)

    return k(table, dst3, srcn3)


def _sc_scatter(mg, dst3, zeros, ne, c):
    epw = ne // _NW
    nch = epw // c
    mesh = plsc.VectorSubcoreMesh(core_axis_name="c", subcore_axis_name="s")

    nbuf = 4

    @functools.partial(
        pl.kernel,
        mesh=mesh,
        out_type=jax.ShapeDtypeStruct((2 * _N, _H), _f32),
        scratch_types=[
            pltpu.VMEM_SHARED((_N, _H), _f32),
            pltpu.VMEM((nbuf, 1, c), jnp.int32),
            pltpu.VMEM((nbuf, c, _H), _f32),
        ] + [pltpu.SemaphoreType.DMA] * (3 * nbuf),
    )
    def k(mg_hbm, dst_hbm, z_hbm, out_hbm, acc_sh, ibuf, mbuf, *sems):
        rd = sems[0:nbuf]
        ri = sems[nbuf:2 * nbuf]
        sc = sems[2 * nbuf:3 * nbuf]
        cc = lax.axis_index("c")
        s = lax.axis_index("s")
        wid = s * _NC + cc

        @pl.when(s < 10)
        def _():
            pltpu.sync_copy(z_hbm.at[pl.ds(s * 1000, 1000)],
                            acc_sh.at[pl.ds(s * 1000, 1000)])

        plsc.subcore_barrier()

        def rows(j):
            return pl.ds(wid * epw + j * c, c)

        def issue_read(j, b):
            pltpu.async_copy(dst_hbm.at[wid, j], ibuf.at[b], ri[b])
            pltpu.async_copy(mg_hbm.at[rows(j)], mbuf.at[b], rd[b])

        issue_read(0, 0)
        issue_read(1, 1)

        def step(i, carry):
            for b in range(nbuf):
                j = i * nbuf + b
                ba = (b + 2) % nbuf

                @pl.when((j >= 2) & (j < nch + 2))
                def _():
                    pltpu.make_async_copy(mbuf.at[ba],
                                          acc_sh.at[ibuf.at[ba, 0]],
                                          sc[ba]).wait()

                @pl.when(j + 2 < nch)
                def _():
                    issue_read(j + 2, ba)

                @pl.when(j < nch)
                def _():
                    pltpu.make_async_copy(mg_hbm.at[rows(j)], mbuf.at[b],
                                          rd[b]).wait()
                    pltpu.make_async_copy(dst_hbm.at[wid, j], ibuf.at[b],
                                          ri[b]).wait()
                    pltpu.async_copy(mbuf.at[b], acc_sh.at[ibuf.at[b, 0]],
                                     sc[b], add=True)
            return carry

        lax.fori_loop(0, (nch + 2 + nbuf - 1) // nbuf, step, 0)
        plsc.subcore_barrier()

        @pl.when(s < 10)
        def _():
            pltpu.sync_copy(acc_sh.at[pl.ds(s * 1000, 1000)],
                            out_hbm.at[pl.ds(cc * _N + s * 1000, 1000)])

    return k(mg, dst3, zeros)


def _tc_project(h, w_stack):
    bn = 1000

    def body(h_ref, w_ref, o_ref):
        o_ref[...] = jnp.dot(h_ref[...], w_ref[0],
                             preferred_element_type=_f32)

    return pl.pallas_call(
        body,
        grid=(2 * _N // bn,),
        in_specs=[
            pl.BlockSpec((bn, _H), lambda g: (g % (_N // bn), 0)),
            pl.BlockSpec((1, _H, _H), lambda g: (g // (_N // bn), 0, 0)),
        ],
        out_specs=pl.BlockSpec((bn, _H), lambda g: (g, 0)),
        out_shape=jax.ShapeDtypeStruct((2 * _N, _H), _f32),
    )(h, w_stack)


def _tc_edge_mlp(attr, g, wa, b1, w2, b2, iw, ib, ne, be):
    def body(a_ref, g_ref, wa_ref, b1_ref, w2_ref, b2_ref, iw_ref,
             ib_ref, o_ref):
        x = (jnp.dot(a_ref[...], wa_ref[...], preferred_element_type=_f32)
             + g_ref[...] + b1_ref[...])
        m = jnp.maximum(x, 0.0)
        mij = jnp.maximum(
            jnp.dot(m, w2_ref[...], preferred_element_type=_f32)
            + b2_ref[...], 0.0)
        t = jnp.sum(mij * iw_ref[...], axis=1, keepdims=True) + ib_ref[0, 0]
        eij = 1.0 / (1.0 + jnp.exp(-t))
        o_ref[...] = mij * eij

    full = lambda g: (0, 0)
    return pl.pallas_call(
        body,
        grid=(ne // be,),
        in_specs=[
            pl.BlockSpec((be, _ED), lambda g: (g, 0)),
            pl.BlockSpec((be, _H), lambda g: (g, 0)),
            pl.BlockSpec((_ED, _H), full),
            pl.BlockSpec((1, _H), full),
            pl.BlockSpec((_H, _H), full),
            pl.BlockSpec((1, _H), full),
            pl.BlockSpec((1, _H), full),
            pl.BlockSpec((1, 1), full),
        ],
        out_specs=pl.BlockSpec((be, _H), lambda g: (g, 0)),
        out_shape=jax.ShapeDtypeStruct((ne, _H), _f32),
    )(attr, g, wa, b1, w2, b2, iw, ib)


def _tc_node_mlp(part_a, part_b, h, wmi, wh, b1, w2, b2):
    bn = 1000

    def body(pa0_ref, pa1_ref, pb0_ref, pb1_ref, h_ref, wmi_ref, wh_ref,
             b1_ref, w2_ref, b2_ref, o_ref):
        mi = (pa0_ref[...] + pa1_ref[...]) + (pb0_ref[...] + pb1_ref[...])
        z = jnp.maximum(
            jnp.dot(mi, wmi_ref[...], preferred_element_type=_f32)
            + jnp.dot(h_ref[...], wh_ref[...], preferred_element_type=_f32)
            + b1_ref[...], 0.0)
        o_ref[...] = jnp.dot(z, w2_ref[...],
                             preferred_element_type=_f32) + b2_ref[...]

    full = lambda g: (0, 0)
    nb = _N // bn
    return pl.pallas_call(
        body,
        grid=(nb,),
        in_specs=[
            pl.BlockSpec((bn, _H), lambda g: (g, 0)),
            pl.BlockSpec((bn, _H), lambda g: (g + nb, 0)),
            pl.BlockSpec((bn, _H), lambda g: (g, 0)),
            pl.BlockSpec((bn, _H), lambda g: (g + nb, 0)),
            pl.BlockSpec((bn, _H), lambda g: (g, 0)),
            pl.BlockSpec((_H, _H), full),
            pl.BlockSpec((_H, _H), full),
            pl.BlockSpec((1, _H), full),
            pl.BlockSpec((_H, _H), full),
            pl.BlockSpec((1, _H), full),
        ],
        out_specs=pl.BlockSpec((bn, _H), lambda g: (g, 0)),
        out_shape=jax.ShapeDtypeStruct((_N, _H), _f32),
    )(part_a, part_a, part_b, part_b, h, wmi, wh, b1, w2, b2)


def kernel(h, edge_index, edge_attr, e_w1, e_b1, e_w2, e_b2, i_w, i_b,
           n_w1, n_b1, n_w2, n_b2):
    dst = edge_index[0].astype(jnp.int32)
    srcn = (edge_index[1] + _N).astype(jnp.int32)

    w_stack = jnp.stack([e_w1[_ED:_ED + _H], e_w1[_ED + _H:]])
    table = _tc_project(h, w_stack)

    zeros = jnp.zeros((_N, _H), _f32)
    wa = e_w1[:_ED]
    b1 = e_b1.reshape(1, _H)
    b2 = e_b2.reshape(1, _H)
    iw = i_w.reshape(1, _H)
    ib = i_b.reshape(1, 1)

    halves = []
    for lo, ne, gc, sc in ((0, _NE0, _GC0, _SC0), (_NE0, _NE1, _GC1, _SC1)):
        epw = ne // _NW
        dh = lax.dynamic_slice_in_dim(dst, lo, ne)
        sh = lax.dynamic_slice_in_dim(srcn, lo, ne)
        halves.append({
            "lo": lo, "ne": ne, "gc": gc, "sc": sc,
            "gd3": dh.reshape(_NW, epw // gc, 1, gc),
            "gs3": sh.reshape(_NW, epw // gc, 1, gc),
            "sd3": dh.reshape(_NW, epw // sc, 1, sc),
        })

    for hv in halves:
        hv["g"] = _sc_gather(table, hv["gd3"], hv["gs3"], hv["ne"], hv["gc"])

    for hv, be in zip(halves, (_BE0, _BE1)):
        attr = lax.dynamic_slice_in_dim(edge_attr, hv["lo"], hv["ne"])
        hv["mg"] = _tc_edge_mlp(attr, hv["g"], wa, b1, e_w2, b2,
                                iw, ib, hv["ne"], be)

    for hv in halves:
        hv["part"] = _sc_scatter(hv["mg"], hv["sd3"], zeros, hv["ne"],
                                 hv["sc"])

    return _tc_node_mlp(halves[0]["part"], halves[1]["part"], h,
                        n_w1[:_H], n_w1[_H:], n_b1.reshape(1, _H),
                        n_w2, n_b2.reshape(1, _H))
